# baseline (speedup 1.0000x reference)
import functools

import jax
import jax.numpy as jnp
from jax import lax
from jax.experimental import pallas as pl
from jax.experimental.pallas import tpu as pltpu
from jax.experimental.pallas import tpu_sc as plsc

N, E, D = 10000, 320000, 128
ROWS_BLK = 1000
N_BLOCKS = N // ROWS_BLK
NTILES = 32
CHUNK = 64
NCHUNK_TOTAL = E // CHUNK
NBASE = NCHUNK_TOTAL // NTILES
NREM = NCHUNK_TOTAL % NTILES
ROWS_PER_SUBCORE = 632
NP = 16 * ROWS_PER_SUBCORE


def _leaky(x):
    return jnp.where(x > 0, x, 0.2 * x)


def _pre_body(x_ref, w1t_ref, b1_ref, w2t_ref, b2_ref, a1_ref, a2_ref,
              ab_ref, xj_ref, ai_ref, aj_ref):
    x = x_ref[...]
    xi = _leaky(jnp.dot(x, w1t_ref[...], preferred_element_type=jnp.float32)
                + b1_ref[...])
    xj = _leaky(jnp.dot(x, w2t_ref[...], preferred_element_type=jnp.float32)
                + b2_ref[...])
    xj_ref[...] = xj
    ai_ref[...] = jnp.sum(xi * a1_ref[...], axis=1, keepdims=True) + ab_ref[0, 0]
    aj_ref[...] = jnp.sum(xj * a2_ref[...], axis=1, keepdims=True) + ab_ref[0, 1]


def _run_pre(x0, w1t, b1r, w2t, b2r, a1r, a2r, abr):
    full = lambda: pl.BlockSpec((1, D), lambda i: (0, 0))
    return pl.pallas_call(
        _pre_body,
        grid=(N_BLOCKS,),
        in_specs=[
            pl.BlockSpec((ROWS_BLK, D), lambda i: (i, 0)),
            pl.BlockSpec((D, D), lambda i: (0, 0)),
            full(),
            pl.BlockSpec((D, D), lambda i: (0, 0)),
            full(), full(), full(), full(),
        ],
        out_specs=[
            pl.BlockSpec((ROWS_BLK, D), lambda i: (i, 0)),
            pl.BlockSpec((ROWS_BLK, 1), lambda i: (i, 0)),
            pl.BlockSpec((ROWS_BLK, 1), lambda i: (i, 0)),
        ],
        out_shape=[
            jax.ShapeDtypeStruct((N, D), jnp.float32),
            jax.ShapeDtypeStruct((N, 1), jnp.float32),
            jax.ShapeDtypeStruct((N, 1), jnp.float32),
        ],
    )(x0, w1t, b1r, w2t, b2r, a1r, a2r, abr)


@functools.partial(
    pl.kernel,
    mesh=plsc.VectorSubcoreMesh(core_axis_name="c", subcore_axis_name="s"),
    out_type=jax.ShapeDtypeStruct((2 * NP, D), jnp.float32),
    compiler_params=pltpu.CompilerParams(needs_layout_passes=False),
    scratch_types=[
        pltpu.VMEM((4, CHUNK), jnp.int32),
        pltpu.VMEM((4, CHUNK), jnp.int32),
        pltpu.VMEM((N,), jnp.float32),
        pltpu.VMEM((N,), jnp.float32),
        pltpu.VMEM((CHUNK,), jnp.float32),
        pltpu.VMEM((3, CHUNK, D), jnp.float32),
        pltpu.VMEM_SHARED((NP, D), jnp.float32),
        pltpu.SemaphoreType.DMA((4,)),
        pltpu.SemaphoreType.DMA((3,)),
        pltpu.SemaphoreType.DMA((3,)),
    ],
)
def _sc_edge_kernel(row_hbm, col_hbm, xj_hbm, ai_hbm, aj_hbm, zeros_hbm,
                    out_hbm, idxr, idxc, ai_l, aj_l, att_v, rows,
                    agg_sh, sem_i, sem_g, sem_s):
    c = lax.axis_index("c")
    s = lax.axis_index("s")
    wid = c * 16 + s
    rslice = pl.ds(s * ROWS_PER_SUBCORE, ROWS_PER_SUBCORE)
    pltpu.sync_copy(zeros_hbm, agg_sh.at[rslice])
    pltpu.sync_copy(ai_hbm, ai_l)
    pltpu.sync_copy(aj_hbm, aj_l)
    plsc.subcore_barrier()

    nt = NBASE + jnp.where(wid < NREM, 1, 0)

    def idx_desc(j):
        b4 = lax.rem(j, 4)
        off = (wid + NTILES * j) * CHUNK
        dr = pltpu.make_async_copy(row_hbm.at[pl.ds(off, CHUNK)],
                                   idxr.at[b4], sem_i.at[b4])
        dc = pltpu.make_async_copy(col_hbm.at[pl.ds(off, CHUNK)],
                                   idxc.at[b4], sem_i.at[b4])
        return dr, dc

    def gather_desc(j):
        b3 = lax.rem(j, 3)
        b4 = lax.rem(j, 4)
        return pltpu.make_async_copy(xj_hbm.at[idxc.at[b4]], rows.at[b3],
                                     sem_g.at[b3])

    def scatter_desc(j):
        b3 = lax.rem(j, 3)
        b4 = lax.rem(j, 4)
        return pltpu.make_async_copy(rows.at[b3], agg_sh.at[idxr.at[b4]],
                                     sem_s.at[b3])

    for d in idx_desc(0):
        d.start()
    for d in idx_desc(1):
        d.start()
    for d in idx_desc(0):
        d.wait()
    gather_desc(0).start()

    def loop_body(j, carry):
        b3 = lax.rem(j, 3)
        b4 = lax.rem(j, 4)
        ir = idxr.at[b4]
        ic = idxc.at[b4]
        for g in range(CHUNK // 16):
            r16 = ir[pl.ds(g * 16, 16)]
            c16 = ic[pl.ds(g * 16, 16)]
            ar = plsc.load_gather(ai_l, [r16])
            ac = plsc.load_gather(aj_l, [c16])
            att_v[pl.ds(g * 16, 16)] = 1.0 / (1.0 + jnp.exp(-(ar + ac)))

        gather_desc(j).wait()

        rb = rows.at[b3]

        def scale_body(e, carry2):
            a16 = plsc.load_gather(att_v, [jnp.full((16,), e, jnp.int32)])
            for q in range(D // 16):
                sl = pl.ds(q * 16, 16)
                rb[e, sl] = rb[e, sl] * a16
            return carry2

        lax.fori_loop(0, CHUNK, scale_body, 0, unroll=2)

        scatter_desc(j).start(add=True)

        @pl.when(j >= 2)
        def _():
            scatter_desc(j - 2).wait()

        @pl.when(j + 2 < nt)
        def _():
            for d in idx_desc(j + 2):
                d.start()

        @pl.when(j + 1 < nt)
        def _():
            for d in idx_desc(j + 1):
                d.wait()
            gather_desc(j + 1).start()

        return carry

    lax.fori_loop(0, nt, loop_body, 0)
    scatter_desc(nt - 2).wait()
    scatter_desc(nt - 1).wait()
    plsc.subcore_barrier()
    pltpu.sync_copy(agg_sh.at[rslice],
                    out_hbm.at[pl.ds(c * NP + s * ROWS_PER_SUBCORE,
                                     ROWS_PER_SUBCORE)])


def _post_body(x_ref, w1t_ref, b1_ref, p_ref, wot_ref, bo_ref, g_ref, be_ref,
               y_ref):
    x = x_ref[...]
    xi = _leaky(jnp.dot(x, w1t_ref[...], preferred_element_type=jnp.float32)
                + b1_ref[...])
    agg = p_ref[0] + p_ref[1]
    agg = 1.0 / (1.0 + jnp.exp(-agg))
    out = (jnp.dot(agg, wot_ref[...], preferred_element_type=jnp.float32)
           + bo_ref[...] + xi + x)
    mean = jnp.mean(out, axis=-1, keepdims=True)
    ctr = out - mean
    var = jnp.mean(ctr * ctr, axis=-1, keepdims=True)
    y_ref[...] = ctr * lax.rsqrt(var + 1e-5) * g_ref[...] + be_ref[...]


def _run_post(x0, w1t, b1r, partials, wot, bor, g1r, be1r):
    full = lambda: pl.BlockSpec((1, D), lambda i: (0, 0))
    return pl.pallas_call(
        _post_body,
        grid=(N_BLOCKS,),
        in_specs=[
            pl.BlockSpec((ROWS_BLK, D), lambda i: (i, 0)),
            pl.BlockSpec((D, D), lambda i: (0, 0)),
            full(),
            pl.BlockSpec((2, ROWS_BLK, D), lambda i: (0, i, 0)),
            pl.BlockSpec((D, D), lambda i: (0, 0)),
            full(), full(), full(),
        ],
        out_specs=pl.BlockSpec((ROWS_BLK, D), lambda i: (i, 0)),
        out_shape=jax.ShapeDtypeStruct((N, D), jnp.float32),
    )(x0, w1t, b1r, partials, wot, bor, g1r, be1r)


def kernel(x0, x1, edge_index, W1, b1, W2, b2, a1w, a1b, a2w, a2b, Wo, bo,
           g1, be1):
    row = edge_index[0]
    col = edge_index[1]
    b1r = b1.reshape(1, D)
    b2r = b2.reshape(1, D)
    abr = jnp.concatenate([a1b, a2b, jnp.zeros((D - 2,), jnp.float32)])
    abr = abr.reshape(1, D)

    xj0, ai0, aj0 = _run_pre(x0, W1.T, b1r, W2.T, b2r, a1w, a2w, abr)

    zeros = jnp.zeros((ROWS_PER_SUBCORE, D), jnp.float32)
    partials = _sc_edge_kernel(row, col, xj0, ai0.reshape(N), aj0.reshape(N),
                               zeros)
    partials = partials.reshape(2, NP, D)[:, :N]

    return _run_post(x0, W1.T, b1r, partials, Wo.T, bo.reshape(1, D),
                     g1.reshape(1, D), be1.reshape(1, D))


# speedup vs baseline: 1.9657x; 1.9657x over previous
import functools

import jax
import jax.numpy as jnp
from jax import lax
from jax.experimental import pallas as pl
from jax.experimental.pallas import tpu as pltpu
from jax.experimental.pallas import tpu_sc as plsc

N, E, D = 10000, 320000, 128
ROWS_BLK = 1000
N_BLOCKS = N // ROWS_BLK
NTILES = 32
CHUNK = 128
NCHUNK_TOTAL = E // CHUNK
NBASE = NCHUNK_TOTAL // NTILES
NREM = NCHUNK_TOTAL % NTILES
ROWS_PER_SUBCORE = 632
NP = 16 * ROWS_PER_SUBCORE


def _leaky(x):
    return jnp.where(x > 0, x, 0.2 * x)


def _pre_body(x_ref, w1t_ref, b1_ref, w2t_ref, b2_ref, a1_ref, a2_ref,
              ab_ref, xj_ref, ai_ref, aj_ref):
    x = x_ref[...]
    xi = _leaky(jnp.dot(x, w1t_ref[...], preferred_element_type=jnp.float32)
                + b1_ref[...])
    xj = _leaky(jnp.dot(x, w2t_ref[...], preferred_element_type=jnp.float32)
                + b2_ref[...])
    xj_ref[...] = xj
    ai_ref[...] = jnp.sum(xi * a1_ref[...], axis=1, keepdims=True) + ab_ref[0, 0]
    aj_ref[...] = jnp.sum(xj * a2_ref[...], axis=1, keepdims=True) + ab_ref[0, 1]


def _run_pre(x0, w1t, b1r, w2t, b2r, a1r, a2r, abr):
    full = lambda: pl.BlockSpec((1, D), lambda i: (0, 0))
    return pl.pallas_call(
        _pre_body,
        grid=(N_BLOCKS,),
        in_specs=[
            pl.BlockSpec((ROWS_BLK, D), lambda i: (i, 0)),
            pl.BlockSpec((D, D), lambda i: (0, 0)),
            full(),
            pl.BlockSpec((D, D), lambda i: (0, 0)),
            full(), full(), full(), full(),
        ],
        out_specs=[
            pl.BlockSpec((ROWS_BLK, D), lambda i: (i, 0)),
            pl.BlockSpec((ROWS_BLK, 1), lambda i: (i, 0)),
            pl.BlockSpec((ROWS_BLK, 1), lambda i: (i, 0)),
        ],
        out_shape=[
            jax.ShapeDtypeStruct((N, D), jnp.float32),
            jax.ShapeDtypeStruct((N, 1), jnp.float32),
            jax.ShapeDtypeStruct((N, 1), jnp.float32),
        ],
    )(x0, w1t, b1r, w2t, b2r, a1r, a2r, abr)


@functools.partial(
    pl.kernel,
    mesh=plsc.VectorSubcoreMesh(core_axis_name="c", subcore_axis_name="s"),
    out_type=[jax.ShapeDtypeStruct((2 * NP, D), jnp.float32),
              jax.ShapeDtypeStruct((E,), jnp.float32)],
    compiler_params=pltpu.CompilerParams(needs_layout_passes=False),
    scratch_types=[
        pltpu.VMEM_SHARED((NP, D), jnp.float32),
        pltpu.SemaphoreType.DMA((4,)),
        pltpu.SemaphoreType.DMA((2,)),
        pltpu.SemaphoreType.DMA((2,)),
        pltpu.SemaphoreType.DMA((4,)),
    ],
)
def _sc_edge_kernel(row_hbm, col_hbm, xj_hbm, ai_hbm, aj_hbm, zeros_hbm,
                    out_hbm, att_hbm, agg_sh, sem_i, sem_g, sem_s, sem_a):
    c = lax.axis_index("c")
    s = lax.axis_index("s")
    wid = c * 16 + s
    rslice = pl.ds(s * ROWS_PER_SUBCORE, ROWS_PER_SUBCORE)
    pltpu.sync_copy(zeros_hbm, agg_sh.at[rslice])

    nt = NBASE + jnp.where(wid < NREM, 1, 0)

    def chunk_off(j):
        return (wid + NTILES * j) * CHUNK

    def phase_a(ai_l, aj_l, idr, idc, attw):
        pltpu.sync_copy(ai_hbm, ai_l)
        pltpu.sync_copy(aj_hbm, aj_l)

        def a_idx_desc(j):
            b = lax.rem(j, 2)
            off = chunk_off(j)
            dr = pltpu.make_async_copy(row_hbm.at[pl.ds(off, CHUNK)],
                                       idr.at[b], sem_i.at[b])
            dc = pltpu.make_async_copy(col_hbm.at[pl.ds(off, CHUNK)],
                                       idc.at[b], sem_i.at[b])
            return dr, dc

        def a_att_desc(j):
            b = lax.rem(j, 2)
            return pltpu.make_async_copy(attw.at[b],
                                         att_hbm.at[pl.ds(chunk_off(j),
                                                          CHUNK)],
                                         sem_a.at[b])

        for d in a_idx_desc(0):
            d.start()

        def a_body(j, carry):
            b = lax.rem(j, 2)
            for d in a_idx_desc(j):
                d.wait()

            @pl.when(j + 1 < nt)
            def _():
                for d in a_idx_desc(j + 1):
                    d.start()

            @pl.when(j >= 2)
            def _():
                a_att_desc(j - 2).wait()

            ir = idr.at[b]
            ic = idc.at[b]
            for g in range(CHUNK // 16):
                r16 = ir[pl.ds(g * 16, 16)]
                c16 = ic[pl.ds(g * 16, 16)]
                ar = plsc.load_gather(ai_l, [r16])
                ac = plsc.load_gather(aj_l, [c16])
                attw[b, pl.ds(g * 16, 16)] = 1.0 / (1.0 + jnp.exp(-(ar + ac)))
            a_att_desc(j).start()
            return carry

        lax.fori_loop(0, nt, a_body, 0)
        a_att_desc(nt - 2).wait()
        a_att_desc(nt - 1).wait()

    pl.run_scoped(
        phase_a,
        pltpu.VMEM((N,), jnp.float32),
        pltpu.VMEM((N,), jnp.float32),
        pltpu.VMEM((2, CHUNK), jnp.int32),
        pltpu.VMEM((2, CHUNK), jnp.int32),
        pltpu.VMEM((2, CHUNK), jnp.float32),
    )

    plsc.subcore_barrier()

    def phase_b(rows, ibr, ibc, attb):
        def b_triple_desc(j):
            b4 = lax.rem(j, 4)
            off = chunk_off(j)
            dr = pltpu.make_async_copy(row_hbm.at[pl.ds(off, CHUNK)],
                                       ibr.at[b4], sem_i.at[b4])
            dc = pltpu.make_async_copy(col_hbm.at[pl.ds(off, CHUNK)],
                                       ibc.at[b4], sem_i.at[b4])
            da = pltpu.make_async_copy(att_hbm.at[pl.ds(off, CHUNK)],
                                       attb.at[b4], sem_a.at[b4])
            return dr, dc, da

        def b_gather_desc(j):
            b2 = lax.rem(j, 2)
            b4 = lax.rem(j, 4)
            return pltpu.make_async_copy(xj_hbm.at[ibc.at[b4]], rows.at[b2],
                                         sem_g.at[b2])

        def b_scatter_desc(j):
            b2 = lax.rem(j, 2)
            b4 = lax.rem(j, 4)
            return pltpu.make_async_copy(rows.at[b2], agg_sh.at[ibr.at[b4]],
                                         sem_s.at[b2])

        for d in b_triple_desc(0):
            d.start()
        for d in b_triple_desc(1):
            d.start()
        for d in b_triple_desc(0):
            d.wait()
        b_gather_desc(0).start()

        def b_body(j, carry):
            b2 = lax.rem(j, 2)
            b4 = lax.rem(j, 4)
            b_gather_desc(j).wait()

            rb = rows.at[b2]

            def scale_body(e, carry2):
                a16 = plsc.load_gather(
                    attb, [jnp.full((16,), b4, jnp.int32),
                           jnp.full((16,), e, jnp.int32)])
                for q in range(D // 16):
                    sl = pl.ds(q * 16, 16)
                    rb[e, sl] = rb[e, sl] * a16
                return carry2

            lax.fori_loop(0, CHUNK, scale_body, 0, unroll=2)

            b_scatter_desc(j).start(add=True)

            @pl.when(j >= 1)
            def _():
                b_scatter_desc(j - 1).wait()

            @pl.when(j + 2 < nt)
            def _():
                for d in b_triple_desc(j + 2):
                    d.start()

            @pl.when(j + 1 < nt)
            def _():
                for d in b_triple_desc(j + 1):
                    d.wait()
                b_gather_desc(j + 1).start()

            return carry

        lax.fori_loop(0, nt, b_body, 0)
        b_scatter_desc(nt - 1).wait()

    pl.run_scoped(
        phase_b,
        pltpu.VMEM((2, CHUNK, D), jnp.float32),
        pltpu.VMEM((4, CHUNK), jnp.int32),
        pltpu.VMEM((4, CHUNK), jnp.int32),
        pltpu.VMEM((4, CHUNK), jnp.float32),
    )

    plsc.subcore_barrier()
    pltpu.sync_copy(agg_sh.at[rslice],
                    out_hbm.at[pl.ds(c * NP + s * ROWS_PER_SUBCORE,
                                     ROWS_PER_SUBCORE)])


def _post_body(x_ref, w1t_ref, b1_ref, p_ref, wot_ref, bo_ref, g_ref, be_ref,
               y_ref):
    x = x_ref[...]
    xi = _leaky(jnp.dot(x, w1t_ref[...], preferred_element_type=jnp.float32)
                + b1_ref[...])
    agg = p_ref[0] + p_ref[1]
    agg = 1.0 / (1.0 + jnp.exp(-agg))
    out = (jnp.dot(agg, wot_ref[...], preferred_element_type=jnp.float32)
           + bo_ref[...] + xi + x)
    mean = jnp.mean(out, axis=-1, keepdims=True)
    ctr = out - mean
    var = jnp.mean(ctr * ctr, axis=-1, keepdims=True)
    y_ref[...] = ctr * lax.rsqrt(var + 1e-5) * g_ref[...] + be_ref[...]


def _run_post(x0, w1t, b1r, partials, wot, bor, g1r, be1r):
    full = lambda: pl.BlockSpec((1, D), lambda i: (0, 0))
    return pl.pallas_call(
        _post_body,
        grid=(N_BLOCKS,),
        in_specs=[
            pl.BlockSpec((ROWS_BLK, D), lambda i: (i, 0)),
            pl.BlockSpec((D, D), lambda i: (0, 0)),
            full(),
            pl.BlockSpec((2, ROWS_BLK, D), lambda i: (0, i, 0)),
            pl.BlockSpec((D, D), lambda i: (0, 0)),
            full(), full(), full(),
        ],
        out_specs=pl.BlockSpec((ROWS_BLK, D), lambda i: (i, 0)),
        out_shape=jax.ShapeDtypeStruct((N, D), jnp.float32),
    )(x0, w1t, b1r, partials, wot, bor, g1r, be1r)


def kernel(x0, x1, edge_index, W1, b1, W2, b2, a1w, a1b, a2w, a2b, Wo, bo,
           g1, be1):
    row = edge_index[0]
    col = edge_index[1]
    b1r = b1.reshape(1, D)
    b2r = b2.reshape(1, D)
    abr = jnp.concatenate([a1b, a2b, jnp.zeros((D - 2,), jnp.float32)])
    abr = abr.reshape(1, D)

    xj0, ai0, aj0 = _run_pre(x0, W1.T, b1r, W2.T, b2r, a1w, a2w, abr)

    zeros = jnp.zeros((ROWS_PER_SUBCORE, D), jnp.float32)
    partials, _att_unused = _sc_edge_kernel(row, col, xj0, ai0.reshape(N),
                                            aj0.reshape(N), zeros)
    partials = partials.reshape(2, NP, D)[:, :N]

    return _run_post(x0, W1.T, b1r, partials, Wo.T, bo.reshape(1, D),
                     g1.reshape(1, D), be1.reshape(1, D))


# speedup vs baseline: 1.9844x; 1.0095x over previous
import functools

import jax
import jax.numpy as jnp
from jax import lax
from jax.experimental import pallas as pl
from jax.experimental.pallas import tpu as pltpu
from jax.experimental.pallas import tpu_sc as plsc

N, E, D = 10000, 320000, 128
ROWS_BLK = 1000
N_BLOCKS = N // ROWS_BLK
NTILES = 32
CHUNK = 128
NCHUNK_TOTAL = E // CHUNK
NBASE = NCHUNK_TOTAL // NTILES
NREM = NCHUNK_TOTAL % NTILES
ROWS_PER_SUBCORE = 632
NP = 16 * ROWS_PER_SUBCORE


def _leaky(x):
    return jnp.where(x > 0, x, 0.2 * x)


def _pre_body(x_ref, w1t_ref, b1_ref, w2t_ref, b2_ref, a1_ref, a2_ref,
              ab_ref, xj_ref, ai_ref, aj_ref):
    x = x_ref[...]
    xi = _leaky(jnp.dot(x, w1t_ref[...], preferred_element_type=jnp.float32)
                + b1_ref[...])
    xj = _leaky(jnp.dot(x, w2t_ref[...], preferred_element_type=jnp.float32)
                + b2_ref[...])
    xj_ref[...] = xj
    ai_ref[...] = jnp.sum(xi * a1_ref[...], axis=1, keepdims=True) + ab_ref[0, 0]
    aj_ref[...] = jnp.sum(xj * a2_ref[...], axis=1, keepdims=True) + ab_ref[0, 1]


def _run_pre(x0, w1t, b1r, w2t, b2r, a1r, a2r, abr):
    full = lambda: pl.BlockSpec((1, D), lambda i: (0, 0))
    return pl.pallas_call(
        _pre_body,
        grid=(N_BLOCKS,),
        in_specs=[
            pl.BlockSpec((ROWS_BLK, D), lambda i: (i, 0)),
            pl.BlockSpec((D, D), lambda i: (0, 0)),
            full(),
            pl.BlockSpec((D, D), lambda i: (0, 0)),
            full(), full(), full(), full(),
        ],
        out_specs=[
            pl.BlockSpec((ROWS_BLK, D), lambda i: (i, 0)),
            pl.BlockSpec((ROWS_BLK, 1), lambda i: (i, 0)),
            pl.BlockSpec((ROWS_BLK, 1), lambda i: (i, 0)),
        ],
        out_shape=[
            jax.ShapeDtypeStruct((N, D), jnp.float32),
            jax.ShapeDtypeStruct((N, 1), jnp.float32),
            jax.ShapeDtypeStruct((N, 1), jnp.float32),
        ],
    )(x0, w1t, b1r, w2t, b2r, a1r, a2r, abr)


@functools.partial(
    pl.kernel,
    mesh=plsc.VectorSubcoreMesh(core_axis_name="c", subcore_axis_name="s"),
    out_type=jax.ShapeDtypeStruct((2 * NP, D), jnp.float32),
    compiler_params=pltpu.CompilerParams(needs_layout_passes=False),
    scratch_types=[
        pltpu.VMEM_SHARED((NP, D), jnp.float32),
        pltpu.VMEM(((NBASE + 1) * CHUNK,), jnp.float32),
        pltpu.SemaphoreType.DMA((4,)),
        pltpu.SemaphoreType.DMA((2,)),
        pltpu.SemaphoreType.DMA((2,)),
    ],
)
def _sc_edge_kernel(row_hbm, col_hbm, xj_hbm, ai_hbm, aj_hbm, zeros_hbm,
                    out_hbm, agg_sh, att_all, sem_i, sem_g, sem_s):
    c = lax.axis_index("c")
    s = lax.axis_index("s")
    wid = c * 16 + s
    rslice = pl.ds(s * ROWS_PER_SUBCORE, ROWS_PER_SUBCORE)
    pltpu.sync_copy(zeros_hbm, agg_sh.at[rslice])

    nt = NBASE + jnp.where(wid < NREM, 1, 0)

    def chunk_off(j):
        return (wid + NTILES * j) * CHUNK

    def phase_a(ai_l, aj_l, idr, idc):
        pltpu.sync_copy(ai_hbm, ai_l)
        pltpu.sync_copy(aj_hbm, aj_l)

        def a_idx_desc(j):
            b = lax.rem(j, 2)
            off = chunk_off(j)
            dr = pltpu.make_async_copy(row_hbm.at[pl.ds(off, CHUNK)],
                                       idr.at[b], sem_i.at[b])
            dc = pltpu.make_async_copy(col_hbm.at[pl.ds(off, CHUNK)],
                                       idc.at[b], sem_i.at[b])
            return dr, dc

        for d in a_idx_desc(0):
            d.start()

        def a_body(j, carry):
            b = lax.rem(j, 2)
            for d in a_idx_desc(j):
                d.wait()

            @pl.when(j + 1 < nt)
            def _():
                for d in a_idx_desc(j + 1):
                    d.start()

            ir = idr.at[b]
            ic = idc.at[b]
            base = j * CHUNK
            for g in range(CHUNK // 16):
                r16 = ir[pl.ds(g * 16, 16)]
                c16 = ic[pl.ds(g * 16, 16)]
                ar = plsc.load_gather(ai_l, [r16])
                ac = plsc.load_gather(aj_l, [c16])
                att_all[pl.ds(base + g * 16, 16)] = (
                    1.0 / (1.0 + jnp.exp(-(ar + ac))))
            return carry

        lax.fori_loop(0, nt, a_body, 0)

    pl.run_scoped(
        phase_a,
        pltpu.VMEM((N,), jnp.float32),
        pltpu.VMEM((N,), jnp.float32),
        pltpu.VMEM((2, CHUNK), jnp.int32),
        pltpu.VMEM((2, CHUNK), jnp.int32),
    )

    plsc.subcore_barrier()

    def phase_b(rows, ibr, ibc):
        def b_idx_desc(j):
            b4 = lax.rem(j, 4)
            off = chunk_off(j)
            dr = pltpu.make_async_copy(row_hbm.at[pl.ds(off, CHUNK)],
                                       ibr.at[b4], sem_i.at[b4])
            dc = pltpu.make_async_copy(col_hbm.at[pl.ds(off, CHUNK)],
                                       ibc.at[b4], sem_i.at[b4])
            return dr, dc

        def b_gather_desc(j):
            b2 = lax.rem(j, 2)
            b4 = lax.rem(j, 4)
            return pltpu.make_async_copy(xj_hbm.at[ibc.at[b4]], rows.at[b2],
                                         sem_g.at[b2])

        def b_scatter_desc(j):
            b2 = lax.rem(j, 2)
            b4 = lax.rem(j, 4)
            return pltpu.make_async_copy(rows.at[b2], agg_sh.at[ibr.at[b4]],
                                         sem_s.at[b2])

        for d in b_idx_desc(0):
            d.start()
        for d in b_idx_desc(1):
            d.start()
        for d in b_idx_desc(0):
            d.wait()
        b_gather_desc(0).start()

        def b_body(j, carry):
            b2 = lax.rem(j, 2)
            b_gather_desc(j).wait()

            rb = rows.at[b2]
            base = j * CHUNK

            def scale_body(e, carry2):
                a16 = plsc.load_gather(
                    att_all, [jnp.full((16,), base + e, jnp.int32)])
                for q in range(D // 16):
                    sl = pl.ds(q * 16, 16)
                    rb[e, sl] = rb[e, sl] * a16
                return carry2

            lax.fori_loop(0, CHUNK, scale_body, 0, unroll=4)

            b_scatter_desc(j).start(add=True)

            @pl.when(j >= 1)
            def _():
                b_scatter_desc(j - 1).wait()

            @pl.when(j + 2 < nt)
            def _():
                for d in b_idx_desc(j + 2):
                    d.start()

            @pl.when(j + 1 < nt)
            def _():
                for d in b_idx_desc(j + 1):
                    d.wait()
                b_gather_desc(j + 1).start()

            return carry

        lax.fori_loop(0, nt, b_body, 0)
        b_scatter_desc(nt - 1).wait()

    pl.run_scoped(
        phase_b,
        pltpu.VMEM((2, CHUNK, D), jnp.float32),
        pltpu.VMEM((4, CHUNK), jnp.int32),
        pltpu.VMEM((4, CHUNK), jnp.int32),
    )

    plsc.subcore_barrier()
    pltpu.sync_copy(agg_sh.at[rslice],
                    out_hbm.at[pl.ds(c * NP + s * ROWS_PER_SUBCORE,
                                     ROWS_PER_SUBCORE)])


def _post_body(x_ref, w1t_ref, b1_ref, p_ref, wot_ref, bo_ref, g_ref, be_ref,
               y_ref):
    x = x_ref[...]
    xi = _leaky(jnp.dot(x, w1t_ref[...], preferred_element_type=jnp.float32)
                + b1_ref[...])
    agg = p_ref[0] + p_ref[1]
    agg = 1.0 / (1.0 + jnp.exp(-agg))
    out = (jnp.dot(agg, wot_ref[...], preferred_element_type=jnp.float32)
           + bo_ref[...] + xi + x)
    mean = jnp.mean(out, axis=-1, keepdims=True)
    ctr = out - mean
    var = jnp.mean(ctr * ctr, axis=-1, keepdims=True)
    y_ref[...] = ctr * lax.rsqrt(var + 1e-5) * g_ref[...] + be_ref[...]


def _run_post(x0, w1t, b1r, partials, wot, bor, g1r, be1r):
    full = lambda: pl.BlockSpec((1, D), lambda i: (0, 0))
    return pl.pallas_call(
        _post_body,
        grid=(N_BLOCKS,),
        in_specs=[
            pl.BlockSpec((ROWS_BLK, D), lambda i: (i, 0)),
            pl.BlockSpec((D, D), lambda i: (0, 0)),
            full(),
            pl.BlockSpec((2, ROWS_BLK, D), lambda i: (0, i, 0)),
            pl.BlockSpec((D, D), lambda i: (0, 0)),
            full(), full(), full(),
        ],
        out_specs=pl.BlockSpec((ROWS_BLK, D), lambda i: (i, 0)),
        out_shape=jax.ShapeDtypeStruct((N, D), jnp.float32),
    )(x0, w1t, b1r, partials, wot, bor, g1r, be1r)


def kernel(x0, x1, edge_index, W1, b1, W2, b2, a1w, a1b, a2w, a2b, Wo, bo,
           g1, be1):
    row = edge_index[0]
    col = edge_index[1]
    b1r = b1.reshape(1, D)
    b2r = b2.reshape(1, D)
    abr = jnp.concatenate([a1b, a2b, jnp.zeros((D - 2,), jnp.float32)])
    abr = abr.reshape(1, D)

    xj0, ai0, aj0 = _run_pre(x0, W1.T, b1r, W2.T, b2r, a1w, a2w, abr)

    zeros = jnp.zeros((ROWS_PER_SUBCORE, D), jnp.float32)
    partials = _sc_edge_kernel(row, col, xj0, ai0.reshape(N),
                               aj0.reshape(N), zeros)
    partials = partials.reshape(2, NP, D)[:, :N]

    return _run_post(x0, W1.T, b1r, partials, Wo.T, bo.reshape(1, D),
                     g1.reshape(1, D), be1.reshape(1, D))


# speedup vs baseline: 2.2106x; 1.1140x over previous
import functools

import jax
import jax.numpy as jnp
from jax import lax
from jax.experimental import pallas as pl
from jax.experimental.pallas import tpu as pltpu
from jax.experimental.pallas import tpu_sc as plsc

N, E, D = 10000, 320000, 128
ROWS_BLK = 1000
N_BLOCKS = N // ROWS_BLK
NTILES = 32
CHUNK = 128
NCHUNK_TOTAL = E // CHUNK
NBASE = NCHUNK_TOTAL // NTILES
NREM = NCHUNK_TOTAL % NTILES
ROWS_PER_SUBCORE = 632
NP = 16 * ROWS_PER_SUBCORE


def _leaky(x):
    return jnp.where(x > 0, x, 0.2 * x)


def _pre_body(x_ref, w1t_ref, b1_ref, w2t_ref, b2_ref, a1_ref, a2_ref,
              ab_ref, xj_ref, ai_ref, aj_ref):
    x = x_ref[...]
    xi = _leaky(jnp.dot(x, w1t_ref[...], preferred_element_type=jnp.float32)
                + b1_ref[...])
    xj = _leaky(jnp.dot(x, w2t_ref[...], preferred_element_type=jnp.float32)
                + b2_ref[...])
    xj_ref[...] = xj
    ai_ref[...] = jnp.sum(xi * a1_ref[...], axis=1, keepdims=True) + ab_ref[0, 0]
    aj_ref[...] = jnp.sum(xj * a2_ref[...], axis=1, keepdims=True) + ab_ref[0, 1]


def _run_pre(x0, w1t, b1r, w2t, b2r, a1r, a2r, abr):
    full = lambda: pl.BlockSpec((1, D), lambda i: (0, 0))
    return pl.pallas_call(
        _pre_body,
        grid=(N_BLOCKS,),
        in_specs=[
            pl.BlockSpec((ROWS_BLK, D), lambda i: (i, 0)),
            pl.BlockSpec((D, D), lambda i: (0, 0)),
            full(),
            pl.BlockSpec((D, D), lambda i: (0, 0)),
            full(), full(), full(), full(),
        ],
        out_specs=[
            pl.BlockSpec((ROWS_BLK, D), lambda i: (i, 0)),
            pl.BlockSpec((ROWS_BLK, 1), lambda i: (i, 0)),
            pl.BlockSpec((ROWS_BLK, 1), lambda i: (i, 0)),
        ],
        out_shape=[
            jax.ShapeDtypeStruct((N, D), jnp.float32),
            jax.ShapeDtypeStruct((N, 1), jnp.float32),
            jax.ShapeDtypeStruct((N, 1), jnp.float32),
        ],
    )(x0, w1t, b1r, w2t, b2r, a1r, a2r, abr)


@functools.partial(
    pl.kernel,
    mesh=plsc.VectorSubcoreMesh(core_axis_name="c", subcore_axis_name="s"),
    out_type=jax.ShapeDtypeStruct((2 * NP, D), jnp.float32),
    compiler_params=pltpu.CompilerParams(needs_layout_passes=False),
    scratch_types=[
        pltpu.VMEM_SHARED((NP, D), jnp.float32),
        pltpu.VMEM((2, CHUNK, D), jnp.float32),
        pltpu.VMEM((4, CHUNK), jnp.int32),
        pltpu.VMEM((4, CHUNK), jnp.int32),
        pltpu.VMEM((4, CHUNK), jnp.float32),
        pltpu.VMEM((4, CHUNK), jnp.float32),
        pltpu.VMEM((CHUNK,), jnp.float32),
        pltpu.SemaphoreType.DMA((4,)),
        pltpu.SemaphoreType.DMA((4,)),
        pltpu.SemaphoreType.DMA((2,)),
        pltpu.SemaphoreType.DMA((2,)),
    ],
)
def _sc_edge_kernel(row_hbm, col_hbm, xj_hbm, ai_hbm, aj_hbm, zeros_hbm,
                    out_hbm, agg_sh, rows, ibr, ibc, arv, acv, att_v,
                    sem_i, sem_l, sem_g, sem_s):
    c = lax.axis_index("c")
    s = lax.axis_index("s")
    wid = c * 16 + s
    rslice = pl.ds(s * ROWS_PER_SUBCORE, ROWS_PER_SUBCORE)
    pltpu.sync_copy(zeros_hbm, agg_sh.at[rslice])

    nt = NBASE + jnp.where(wid < NREM, 1, 0)

    def chunk_off(j):
        return (wid + NTILES * j) * CHUNK

    def idx_desc(j):
        b4 = lax.rem(j, 4)
        off = chunk_off(j)
        dr = pltpu.make_async_copy(row_hbm.at[pl.ds(off, CHUNK)],
                                   ibr.at[b4], sem_i.at[b4])
        dc = pltpu.make_async_copy(col_hbm.at[pl.ds(off, CHUNK)],
                                   ibc.at[b4], sem_i.at[b4])
        return dr, dc

    def logit_desc(j):
        b4 = lax.rem(j, 4)
        dr = pltpu.make_async_copy(ai_hbm.at[ibr.at[b4]], arv.at[b4],
                                   sem_l.at[b4])
        dc = pltpu.make_async_copy(aj_hbm.at[ibc.at[b4]], acv.at[b4],
                                   sem_l.at[b4])
        return dr, dc

    def gather_desc(j):
        b2 = lax.rem(j, 2)
        b4 = lax.rem(j, 4)
        return pltpu.make_async_copy(xj_hbm.at[ibc.at[b4]], rows.at[b2],
                                     sem_g.at[b2])

    def scatter_desc(j):
        b2 = lax.rem(j, 2)
        b4 = lax.rem(j, 4)
        return pltpu.make_async_copy(rows.at[b2], agg_sh.at[ibr.at[b4]],
                                     sem_s.at[b2])

    plsc.subcore_barrier()

    for d in idx_desc(0):
        d.start()
    for d in idx_desc(1):
        d.start()
    for d in idx_desc(0):
        d.wait()
    for d in logit_desc(0):
        d.start()
    gather_desc(0).start()

    def body(j, carry):
        b2 = lax.rem(j, 2)
        b4 = lax.rem(j, 4)

        for d in logit_desc(j):
            d.wait()
        ar_r = arv.at[b4]
        ac_r = acv.at[b4]
        for g in range(CHUNK // 16):
            sl = pl.ds(g * 16, 16)
            att_v[sl] = 1.0 / (1.0 + jnp.exp(-(ar_r[sl] + ac_r[sl])))

        gather_desc(j).wait()

        rb = rows.at[b2]

        def scale_body(e, carry2):
            a16 = plsc.load_gather(att_v, [jnp.full((16,), e, jnp.int32)])
            for q in range(D // 16):
                sl = pl.ds(q * 16, 16)
                rb[e, sl] = rb[e, sl] * a16
            return carry2

        lax.fori_loop(0, CHUNK, scale_body, 0, unroll=4)

        scatter_desc(j).start(add=True)

        @pl.when(j >= 1)
        def _():
            scatter_desc(j - 1).wait()

        @pl.when(j + 2 < nt)
        def _():
            for d in idx_desc(j + 2):
                d.start()

        @pl.when(j + 1 < nt)
        def _():
            for d in idx_desc(j + 1):
                d.wait()
            for d in logit_desc(j + 1):
                d.start()
            gather_desc(j + 1).start()

        return carry

    lax.fori_loop(0, nt, body, 0)
    scatter_desc(nt - 1).wait()

    plsc.subcore_barrier()
    pltpu.sync_copy(agg_sh.at[rslice],
                    out_hbm.at[pl.ds(c * NP + s * ROWS_PER_SUBCORE,
                                     ROWS_PER_SUBCORE)])


def _post_body(x_ref, w1t_ref, b1_ref, p_ref, wot_ref, bo_ref, g_ref, be_ref,
               y_ref):
    x = x_ref[...]
    xi = _leaky(jnp.dot(x, w1t_ref[...], preferred_element_type=jnp.float32)
                + b1_ref[...])
    agg = p_ref[0] + p_ref[1]
    agg = 1.0 / (1.0 + jnp.exp(-agg))
    out = (jnp.dot(agg, wot_ref[...], preferred_element_type=jnp.float32)
           + bo_ref[...] + xi + x)
    mean = jnp.mean(out, axis=-1, keepdims=True)
    ctr = out - mean
    var = jnp.mean(ctr * ctr, axis=-1, keepdims=True)
    y_ref[...] = ctr * lax.rsqrt(var + 1e-5) * g_ref[...] + be_ref[...]


def _run_post(x0, w1t, b1r, partials, wot, bor, g1r, be1r):
    full = lambda: pl.BlockSpec((1, D), lambda i: (0, 0))
    return pl.pallas_call(
        _post_body,
        grid=(N_BLOCKS,),
        in_specs=[
            pl.BlockSpec((ROWS_BLK, D), lambda i: (i, 0)),
            pl.BlockSpec((D, D), lambda i: (0, 0)),
            full(),
            pl.BlockSpec((2, ROWS_BLK, D), lambda i: (0, i, 0)),
            pl.BlockSpec((D, D), lambda i: (0, 0)),
            full(), full(), full(),
        ],
        out_specs=pl.BlockSpec((ROWS_BLK, D), lambda i: (i, 0)),
        out_shape=jax.ShapeDtypeStruct((N, D), jnp.float32),
    )(x0, w1t, b1r, partials, wot, bor, g1r, be1r)


def kernel(x0, x1, edge_index, W1, b1, W2, b2, a1w, a1b, a2w, a2b, Wo, bo,
           g1, be1):
    row = edge_index[0]
    col = edge_index[1]
    b1r = b1.reshape(1, D)
    b2r = b2.reshape(1, D)
    abr = jnp.concatenate([a1b, a2b, jnp.zeros((D - 2,), jnp.float32)])
    abr = abr.reshape(1, D)

    xj0, ai0, aj0 = _run_pre(x0, W1.T, b1r, W2.T, b2r, a1w, a2w, abr)

    zeros = jnp.zeros((ROWS_PER_SUBCORE, D), jnp.float32)
    partials = _sc_edge_kernel(row, col, xj0, ai0.reshape(N),
                               aj0.reshape(N), zeros)
    partials = partials.reshape(2, NP, D)[:, :N]

    return _run_post(x0, W1.T, b1r, partials, Wo.T, bo.reshape(1, D),
                     g1.reshape(1, D), be1.reshape(1, D))


# speedup vs baseline: 2.8969x; 1.3104x over previous
import functools

import jax
import jax.numpy as jnp
from jax import lax
from jax.experimental import pallas as pl
from jax.experimental.pallas import tpu as pltpu
from jax.experimental.pallas import tpu_sc as plsc

N, E, D = 10000, 320000, 128
ROWS_BLK = 1000
N_BLOCKS = N // ROWS_BLK
NTILES = 32
CHUNK = 128
NCHUNK_TOTAL = E // CHUNK
NBASE = NCHUNK_TOTAL // NTILES
NREM = NCHUNK_TOTAL % NTILES
ROWS_PER_SUBCORE = 632
NP = 16 * ROWS_PER_SUBCORE


def _leaky(x):
    return jnp.where(x > 0, x, 0.2 * x)


def _pre_body(x_ref, w1t_ref, b1_ref, w2t_ref, b2_ref, a1_ref, a2_ref,
              ab_ref, xj_ref, ai_ref, aj_ref):
    x = x_ref[...]
    xi = _leaky(jnp.dot(x, w1t_ref[...], preferred_element_type=jnp.float32)
                + b1_ref[...])
    xj = _leaky(jnp.dot(x, w2t_ref[...], preferred_element_type=jnp.float32)
                + b2_ref[...])
    xj_ref[...] = xj
    ai_ref[...] = jnp.sum(xi * a1_ref[...], axis=1, keepdims=True) + ab_ref[0, 0]
    aj_ref[...] = jnp.sum(xj * a2_ref[...], axis=1, keepdims=True) + ab_ref[0, 1]


def _run_pre(x0, w1t, b1r, w2t, b2r, a1r, a2r, abr):
    full = lambda: pl.BlockSpec((1, D), lambda i: (0, 0))
    return pl.pallas_call(
        _pre_body,
        grid=(N_BLOCKS,),
        in_specs=[
            pl.BlockSpec((ROWS_BLK, D), lambda i: (i, 0)),
            pl.BlockSpec((D, D), lambda i: (0, 0)),
            full(),
            pl.BlockSpec((D, D), lambda i: (0, 0)),
            full(), full(), full(), full(),
        ],
        out_specs=[
            pl.BlockSpec((ROWS_BLK, D), lambda i: (i, 0)),
            pl.BlockSpec((ROWS_BLK, 1), lambda i: (i, 0)),
            pl.BlockSpec((ROWS_BLK, 1), lambda i: (i, 0)),
        ],
        out_shape=[
            jax.ShapeDtypeStruct((N, D), jnp.float32),
            jax.ShapeDtypeStruct((N, 1), jnp.float32),
            jax.ShapeDtypeStruct((N, 1), jnp.float32),
        ],
    )(x0, w1t, b1r, w2t, b2r, a1r, a2r, abr)


@functools.partial(
    pl.kernel,
    mesh=plsc.VectorSubcoreMesh(core_axis_name="c", subcore_axis_name="s"),
    out_type=jax.ShapeDtypeStruct((2 * NP, D), jnp.float32),
    compiler_params=pltpu.CompilerParams(needs_layout_passes=False),
    scratch_types=[
        pltpu.VMEM_SHARED((NP, D), jnp.float32),
        pltpu.VMEM((2, CHUNK, D), jnp.float32),
        pltpu.VMEM((4, 2, CHUNK // 2), jnp.int32),
        pltpu.VMEM((2, CHUNK), jnp.int32),
        pltpu.VMEM((2, 2, CHUNK // 2), jnp.float32),
        pltpu.VMEM((2, CHUNK), jnp.float32),
        pltpu.VMEM((CHUNK,), jnp.float32),
        pltpu.SemaphoreType.DMA((4,)),
        pltpu.SemaphoreType.DMA((2,)),
        pltpu.SemaphoreType.DMA((2,)),
        pltpu.SemaphoreType.DMA((2,)),
    ],
)
def _sc_edge_kernel(row_hbm, col_hbm, xj_hbm, ai_hbm, aj_hbm, zeros_hbm,
                    out_hbm, agg_sh, rows, ibr, ibc, arv, acv, att_v,
                    sem_i, sem_l, sem_g, sem_s):
    c = lax.axis_index("c")
    s = lax.axis_index("s")
    wid = c * 16 + s
    HALF = CHUNK // 2
    rslice = pl.ds(s * ROWS_PER_SUBCORE, ROWS_PER_SUBCORE)
    pltpu.sync_copy(zeros_hbm, agg_sh.at[rslice])

    nt = NBASE + jnp.where(wid < NREM, 1, 0)

    def chunk_off(j):
        return (wid + NTILES * j) * CHUNK

    def idx_desc(j):
        b4 = lax.rem(j, 4)
        b2 = lax.rem(j, 2)
        off = chunk_off(j)
        d0 = pltpu.make_async_copy(row_hbm.at[pl.ds(off, HALF)],
                                   ibr.at[b4, 0], sem_i.at[b4])
        d1 = pltpu.make_async_copy(row_hbm.at[pl.ds(off + HALF, HALF)],
                                   ibr.at[b4, 1], sem_i.at[b4])
        dc = pltpu.make_async_copy(col_hbm.at[pl.ds(off, CHUNK)],
                                   ibc.at[b2], sem_i.at[b4])
        return d0, d1, dc

    def logit_desc(j):
        b4 = lax.rem(j, 4)
        b2 = lax.rem(j, 2)
        d0 = pltpu.make_async_copy(ai_hbm.at[ibr.at[b4, 0]], arv.at[b2, 0],
                                   sem_l.at[b2])
        d1 = pltpu.make_async_copy(ai_hbm.at[ibr.at[b4, 1]], arv.at[b2, 1],
                                   sem_l.at[b2])
        dc = pltpu.make_async_copy(aj_hbm.at[ibc.at[b2]], acv.at[b2],
                                   sem_l.at[b2])
        return d0, d1, dc

    def gather_desc(j):
        b2 = lax.rem(j, 2)
        return pltpu.make_async_copy(xj_hbm.at[ibc.at[b2]], rows.at[b2],
                                     sem_g.at[b2])

    def scatter_desc(j, h):
        b2 = lax.rem(j, 2)
        b4 = lax.rem(j, 4)
        return pltpu.make_async_copy(rows.at[b2, pl.ds(h * HALF, HALF)],
                                     agg_sh.at[ibr.at[b4, h]],
                                     sem_s.at[b2])

    plsc.subcore_barrier()

    for d in idx_desc(0):
        d.start()
    for d in idx_desc(1):
        d.start()
    for d in idx_desc(0):
        d.wait()
    for d in logit_desc(0):
        d.start()
    gather_desc(0).start()

    def body(j, carry):
        b2 = lax.rem(j, 2)

        for d in logit_desc(j):
            d.wait()
        ar_r = arv.at[b2]
        ac_r = acv.at[b2]
        for g in range(CHUNK // 16):
            h, o = divmod(g * 16, HALF)
            ar16 = ar_r[h, pl.ds(o, 16)]
            ac16 = ac_r[pl.ds(g * 16, 16)]
            att_v[pl.ds(g * 16, 16)] = 1.0 / (1.0 + jnp.exp(-(ar16 + ac16)))

        gather_desc(j).wait()

        @pl.when(j >= 1)
        def _():
            for h in range(2):
                scatter_desc(j - 1, h).wait()

        @pl.when(j + 2 < nt)
        def _():
            for d in idx_desc(j + 2):
                d.start()

        @pl.when(j + 1 < nt)
        def _():
            for d in idx_desc(j + 1):
                d.wait()
            for d in logit_desc(j + 1):
                d.start()
            gather_desc(j + 1).start()

        rb = rows.at[b2]

        def scale_body(e, carry2):
            a16 = plsc.load_gather(att_v, [jnp.full((16,), e, jnp.int32)])
            for q in range(D // 16):
                sl = pl.ds(q * 16, 16)
                rb[e, sl] = rb[e, sl] * a16
            return carry2

        lax.fori_loop(0, HALF, scale_body, 0, unroll=4)
        scatter_desc(j, 0).start(add=True)
        lax.fori_loop(HALF, CHUNK, scale_body, 0, unroll=4)
        scatter_desc(j, 1).start(add=True)

        return carry

    lax.fori_loop(0, nt, body, 0)
    for h in range(2):
        scatter_desc(nt - 1, h).wait()

    plsc.subcore_barrier()
    pltpu.sync_copy(agg_sh.at[rslice],
                    out_hbm.at[pl.ds(c * NP + s * ROWS_PER_SUBCORE,
                                     ROWS_PER_SUBCORE)])


def _post_body(x_ref, w1t_ref, b1_ref, p_ref, wot_ref, bo_ref, g_ref, be_ref,
               y_ref):
    x = x_ref[...]
    xi = _leaky(jnp.dot(x, w1t_ref[...], preferred_element_type=jnp.float32)
                + b1_ref[...])
    agg = p_ref[0] + p_ref[1]
    agg = 1.0 / (1.0 + jnp.exp(-agg))
    out = (jnp.dot(agg, wot_ref[...], preferred_element_type=jnp.float32)
           + bo_ref[...] + xi + x)
    mean = jnp.mean(out, axis=-1, keepdims=True)
    ctr = out - mean
    var = jnp.mean(ctr * ctr, axis=-1, keepdims=True)
    y_ref[...] = ctr * lax.rsqrt(var + 1e-5) * g_ref[...] + be_ref[...]


def _run_post(x0, w1t, b1r, partials, wot, bor, g1r, be1r):
    full = lambda: pl.BlockSpec((1, D), lambda i: (0, 0))
    return pl.pallas_call(
        _post_body,
        grid=(N_BLOCKS,),
        in_specs=[
            pl.BlockSpec((ROWS_BLK, D), lambda i: (i, 0)),
            pl.BlockSpec((D, D), lambda i: (0, 0)),
            full(),
            pl.BlockSpec((2, ROWS_BLK, D), lambda i: (0, i, 0)),
            pl.BlockSpec((D, D), lambda i: (0, 0)),
            full(), full(), full(),
        ],
        out_specs=pl.BlockSpec((ROWS_BLK, D), lambda i: (i, 0)),
        out_shape=jax.ShapeDtypeStruct((N, D), jnp.float32),
    )(x0, w1t, b1r, partials, wot, bor, g1r, be1r)


def kernel(x0, x1, edge_index, W1, b1, W2, b2, a1w, a1b, a2w, a2b, Wo, bo,
           g1, be1):
    row = edge_index[0]
    col = edge_index[1]
    b1r = b1.reshape(1, D)
    b2r = b2.reshape(1, D)
    abr = jnp.concatenate([a1b, a2b, jnp.zeros((D - 2,), jnp.float32)])
    abr = abr.reshape(1, D)

    xj0, ai0, aj0 = _run_pre(x0, W1.T, b1r, W2.T, b2r, a1w, a2w, abr)

    zeros = jnp.zeros((ROWS_PER_SUBCORE, D), jnp.float32)
    partials = _sc_edge_kernel(row, col, xj0, ai0.reshape(N),
                               aj0.reshape(N), zeros)
    partials = partials.reshape(2, NP, D)[:, :N]

    return _run_post(x0, W1.T, b1r, partials, Wo.T, bo.reshape(1, D),
                     g1.reshape(1, D), be1.reshape(1, D))


# speedup vs baseline: 2.8985x; 1.0005x over previous
import functools

import jax
import jax.numpy as jnp
from jax import lax
from jax.experimental import pallas as pl
from jax.experimental.pallas import tpu as pltpu
from jax.experimental.pallas import tpu_sc as plsc

N, E, D = 10000, 320000, 128
ROWS_BLK = 1000
N_BLOCKS = N // ROWS_BLK
NTILES = 32
CHUNK = 128
NCHUNK_TOTAL = E // CHUNK
NBASE = NCHUNK_TOTAL // NTILES
NREM = NCHUNK_TOTAL % NTILES
ROWS_PER_SUBCORE = 632
NP = 16 * ROWS_PER_SUBCORE


def _leaky(x):
    return jnp.where(x > 0, x, 0.2 * x)


def _pre_body(x_ref, w1t_ref, b1_ref, w2t_ref, b2_ref, a1_ref, a2_ref,
              ab_ref, xj_ref, ai_ref, aj_ref):
    x = x_ref[...]
    xi = _leaky(jnp.dot(x, w1t_ref[...], preferred_element_type=jnp.float32)
                + b1_ref[...])
    xj = _leaky(jnp.dot(x, w2t_ref[...], preferred_element_type=jnp.float32)
                + b2_ref[...])
    xj_ref[...] = xj
    ai_ref[...] = jnp.sum(xi * a1_ref[...], axis=1, keepdims=True) + ab_ref[0, 0]
    aj_ref[...] = jnp.sum(xj * a2_ref[...], axis=1, keepdims=True) + ab_ref[0, 1]


def _run_pre(x0, w1t, b1r, w2t, b2r, a1r, a2r, abr):
    full = lambda: pl.BlockSpec((1, D), lambda i: (0, 0))
    return pl.pallas_call(
        _pre_body,
        grid=(N_BLOCKS,),
        in_specs=[
            pl.BlockSpec((ROWS_BLK, D), lambda i: (i, 0)),
            pl.BlockSpec((D, D), lambda i: (0, 0)),
            full(),
            pl.BlockSpec((D, D), lambda i: (0, 0)),
            full(), full(), full(), full(),
        ],
        out_specs=[
            pl.BlockSpec((ROWS_BLK, D), lambda i: (i, 0)),
            pl.BlockSpec((ROWS_BLK, 1), lambda i: (i, 0)),
            pl.BlockSpec((ROWS_BLK, 1), lambda i: (i, 0)),
        ],
        out_shape=[
            jax.ShapeDtypeStruct((N, D), jnp.float32),
            jax.ShapeDtypeStruct((N, 1), jnp.float32),
            jax.ShapeDtypeStruct((N, 1), jnp.float32),
        ],
    )(x0, w1t, b1r, w2t, b2r, a1r, a2r, abr)


@functools.partial(
    pl.kernel,
    mesh=plsc.VectorSubcoreMesh(core_axis_name="c", subcore_axis_name="s"),
    out_type=jax.ShapeDtypeStruct((2 * NP, D), jnp.float32),
    compiler_params=pltpu.CompilerParams(needs_layout_passes=False),
    scratch_types=[
        pltpu.VMEM_SHARED((NP, D), jnp.float32),
        pltpu.VMEM((2, CHUNK, D), jnp.float32),
        pltpu.VMEM((4, 2, CHUNK // 2), jnp.int32),
        pltpu.VMEM((2, CHUNK), jnp.int32),
        pltpu.VMEM((2, 2, CHUNK // 2), jnp.float32),
        pltpu.VMEM((2, CHUNK), jnp.float32),
        pltpu.VMEM((CHUNK,), jnp.float32),
        pltpu.SemaphoreType.DMA((4,)),
        pltpu.SemaphoreType.DMA((2,)),
        pltpu.SemaphoreType.DMA((2,)),
        pltpu.SemaphoreType.DMA((2,)),
    ],
)
def _sc_edge_kernel(row_hbm, col_hbm, xj_hbm, ai_hbm, aj_hbm, zeros_hbm,
                    out_hbm, agg_sh, rows, ibr, ibc, arv, acv, att_v,
                    sem_i, sem_l, sem_g, sem_s):
    c = lax.axis_index("c")
    s = lax.axis_index("s")
    wid = c * 16 + s
    HALF = CHUNK // 2
    rslice = pl.ds(s * ROWS_PER_SUBCORE, ROWS_PER_SUBCORE)
    pltpu.sync_copy(zeros_hbm, agg_sh.at[rslice])

    nt = NBASE + jnp.where(wid < NREM, 1, 0)

    def chunk_off(j):
        return (wid + NTILES * j) * CHUNK

    def idx_desc(j):
        b4 = lax.rem(j, 4)
        b2 = lax.rem(j, 2)
        off = chunk_off(j)
        d0 = pltpu.make_async_copy(row_hbm.at[pl.ds(off, HALF)],
                                   ibr.at[b4, 0], sem_i.at[b4])
        d1 = pltpu.make_async_copy(row_hbm.at[pl.ds(off + HALF, HALF)],
                                   ibr.at[b4, 1], sem_i.at[b4])
        dc = pltpu.make_async_copy(col_hbm.at[pl.ds(off, CHUNK)],
                                   ibc.at[b2], sem_i.at[b4])
        return d0, d1, dc

    def logit_desc(j):
        b4 = lax.rem(j, 4)
        b2 = lax.rem(j, 2)
        d0 = pltpu.make_async_copy(ai_hbm.at[ibr.at[b4, 0]], arv.at[b2, 0],
                                   sem_l.at[b2])
        d1 = pltpu.make_async_copy(ai_hbm.at[ibr.at[b4, 1]], arv.at[b2, 1],
                                   sem_l.at[b2])
        dc = pltpu.make_async_copy(aj_hbm.at[ibc.at[b2]], acv.at[b2],
                                   sem_l.at[b2])
        return d0, d1, dc

    def gather_desc(j):
        b2 = lax.rem(j, 2)
        return pltpu.make_async_copy(xj_hbm.at[ibc.at[b2]], rows.at[b2],
                                     sem_g.at[b2])

    def scatter_desc(j, h):
        b2 = lax.rem(j, 2)
        b4 = lax.rem(j, 4)
        return pltpu.make_async_copy(rows.at[b2, pl.ds(h * HALF, HALF)],
                                     agg_sh.at[ibr.at[b4, h]],
                                     sem_s.at[b2])

    plsc.subcore_barrier()

    for d in idx_desc(0):
        d.start()
    for d in idx_desc(1):
        d.start()
    for d in idx_desc(0):
        d.wait()
    for d in logit_desc(0):
        d.start()
    gather_desc(0).start()

    def body(j, carry):
        b2 = lax.rem(j, 2)

        for d in logit_desc(j):
            d.wait()
        ar_r = arv.at[b2]
        ac_r = acv.at[b2]
        for g in range(CHUNK // 16):
            h, o = divmod(g * 16, HALF)
            ar16 = ar_r[h, pl.ds(o, 16)]
            ac16 = ac_r[pl.ds(g * 16, 16)]
            att_v[pl.ds(g * 16, 16)] = 1.0 / (1.0 + jnp.exp(-(ar16 + ac16)))

        gather_desc(j).wait()

        @pl.when(j >= 1)
        def _():
            for h in range(2):
                scatter_desc(j - 1, h).wait()

        @pl.when(j + 2 < nt)
        def _():
            for d in idx_desc(j + 2):
                d.start()

        @pl.when(j + 1 < nt)
        def _():
            for d in idx_desc(j + 1):
                d.wait()
            for d in logit_desc(j + 1):
                d.start()
            gather_desc(j + 1).start()

        rb = rows.at[b2]

        def scale_body(e, carry2):
            a16 = plsc.load_gather(att_v, [jnp.full((16,), e, jnp.int32)])
            for q in range(D // 16):
                sl = pl.ds(q * 16, 16)
                rb[e, sl] = rb[e, sl] * a16
            return carry2

        lax.fori_loop(0, HALF, scale_body, 0, unroll=8)
        scatter_desc(j, 0).start(add=True)
        lax.fori_loop(HALF, CHUNK, scale_body, 0, unroll=8)
        scatter_desc(j, 1).start(add=True)

        return carry

    lax.fori_loop(0, nt, body, 0)
    for h in range(2):
        scatter_desc(nt - 1, h).wait()

    plsc.subcore_barrier()
    pltpu.sync_copy(agg_sh.at[rslice],
                    out_hbm.at[pl.ds(c * NP + s * ROWS_PER_SUBCORE,
                                     ROWS_PER_SUBCORE)])


def _post_body(x_ref, w1t_ref, b1_ref, p_ref, wot_ref, bo_ref, g_ref, be_ref,
               y_ref):
    x = x_ref[...]
    xi = _leaky(jnp.dot(x, w1t_ref[...], preferred_element_type=jnp.float32)
                + b1_ref[...])
    agg = p_ref[0] + p_ref[1]
    agg = 1.0 / (1.0 + jnp.exp(-agg))
    out = (jnp.dot(agg, wot_ref[...], preferred_element_type=jnp.float32)
           + bo_ref[...] + xi + x)
    mean = jnp.mean(out, axis=-1, keepdims=True)
    ctr = out - mean
    var = jnp.mean(ctr * ctr, axis=-1, keepdims=True)
    y_ref[...] = ctr * lax.rsqrt(var + 1e-5) * g_ref[...] + be_ref[...]


def _run_post(x0, w1t, b1r, partials, wot, bor, g1r, be1r):
    full = lambda: pl.BlockSpec((1, D), lambda i: (0, 0))
    return pl.pallas_call(
        _post_body,
        grid=(N_BLOCKS,),
        in_specs=[
            pl.BlockSpec((ROWS_BLK, D), lambda i: (i, 0)),
            pl.BlockSpec((D, D), lambda i: (0, 0)),
            full(),
            pl.BlockSpec((2, ROWS_BLK, D), lambda i: (0, i, 0)),
            pl.BlockSpec((D, D), lambda i: (0, 0)),
            full(), full(), full(),
        ],
        out_specs=pl.BlockSpec((ROWS_BLK, D), lambda i: (i, 0)),
        out_shape=jax.ShapeDtypeStruct((N, D), jnp.float32),
    )(x0, w1t, b1r, partials, wot, bor, g1r, be1r)


def kernel(x0, x1, edge_index, W1, b1, W2, b2, a1w, a1b, a2w, a2b, Wo, bo,
           g1, be1):
    row = edge_index[0]
    col = edge_index[1]
    b1r = b1.reshape(1, D)
    b2r = b2.reshape(1, D)
    abr = jnp.concatenate([a1b, a2b, jnp.zeros((D - 2,), jnp.float32)])
    abr = abr.reshape(1, D)

    xj0, ai0, aj0 = _run_pre(x0, W1.T, b1r, W2.T, b2r, a1w, a2w, abr)

    zeros = jnp.zeros((ROWS_PER_SUBCORE, D), jnp.float32)
    partials = _sc_edge_kernel(row, col, xj0, ai0.reshape(N),
                               aj0.reshape(N), zeros)
    partials = partials.reshape(2, NP, D)[:, :N]

    return _run_post(x0, W1.T, b1r, partials, Wo.T, bo.reshape(1, D),
                     g1.reshape(1, D), be1.reshape(1, D))


# speedup vs baseline: 2.9238x; 1.0087x over previous
import functools

import jax
import jax.numpy as jnp
from jax import lax
from jax.experimental import pallas as pl
from jax.experimental.pallas import tpu as pltpu
from jax.experimental.pallas import tpu_sc as plsc

N, E, D = 10000, 320000, 128
ROWS_BLK = 1000
N_BLOCKS = N // ROWS_BLK
NTILES = 32
CHUNK = 128
NCHUNK_TOTAL = E // CHUNK
NBASE = NCHUNK_TOTAL // NTILES
NREM = NCHUNK_TOTAL % NTILES
ROWS_PER_SUBCORE = 632
NP = 16 * ROWS_PER_SUBCORE


def _leaky(x):
    return jnp.where(x > 0, x, 0.2 * x)


def _pre_body(x_ref, w1t_ref, b1_ref, w2t_ref, b2_ref, a1_ref, a2_ref,
              ab_ref, xj_ref, ai_ref, aj_ref):
    x = x_ref[...]
    xi = _leaky(jnp.dot(x, w1t_ref[...], preferred_element_type=jnp.float32)
                + b1_ref[...])
    xj = _leaky(jnp.dot(x, w2t_ref[...], preferred_element_type=jnp.float32)
                + b2_ref[...])
    xj_ref[...] = xj
    ai_ref[...] = jnp.sum(xi * a1_ref[...], axis=1, keepdims=True) + ab_ref[0, 0]
    aj_ref[...] = jnp.sum(xj * a2_ref[...], axis=1, keepdims=True) + ab_ref[0, 1]


def _run_pre(x0, w1t, b1r, w2t, b2r, a1r, a2r, abr):
    full = lambda: pl.BlockSpec((1, D), lambda i: (0, 0))
    return pl.pallas_call(
        _pre_body,
        grid=(N_BLOCKS,),
        in_specs=[
            pl.BlockSpec((ROWS_BLK, D), lambda i: (i, 0)),
            pl.BlockSpec((D, D), lambda i: (0, 0)),
            full(),
            pl.BlockSpec((D, D), lambda i: (0, 0)),
            full(), full(), full(), full(),
        ],
        out_specs=[
            pl.BlockSpec((ROWS_BLK, D), lambda i: (i, 0)),
            pl.BlockSpec((ROWS_BLK, 1), lambda i: (i, 0)),
            pl.BlockSpec((ROWS_BLK, 1), lambda i: (i, 0)),
        ],
        out_shape=[
            jax.ShapeDtypeStruct((N, D), jnp.float32),
            jax.ShapeDtypeStruct((N, 1), jnp.float32),
            jax.ShapeDtypeStruct((N, 1), jnp.float32),
        ],
    )(x0, w1t, b1r, w2t, b2r, a1r, a2r, abr)


@functools.partial(
    pl.kernel,
    mesh=plsc.VectorSubcoreMesh(core_axis_name="c", subcore_axis_name="s"),
    out_type=jax.ShapeDtypeStruct((2 * NP, D), jnp.float32),
    compiler_params=pltpu.CompilerParams(needs_layout_passes=False),
    scratch_types=[
        pltpu.VMEM_SHARED((NP, D), jnp.float32),
        pltpu.VMEM((2, CHUNK, D), jnp.float32),
        pltpu.VMEM((4, 2, CHUNK // 2), jnp.int32),
        pltpu.VMEM((2, CHUNK), jnp.int32),
        pltpu.VMEM((N,), jnp.float32),
        pltpu.VMEM((2, CHUNK), jnp.float32),
        pltpu.VMEM((CHUNK,), jnp.float32),
        pltpu.SemaphoreType.DMA((4,)),
        pltpu.SemaphoreType.DMA((2,)),
        pltpu.SemaphoreType.DMA((2,)),
        pltpu.SemaphoreType.DMA((2,)),
    ],
)
def _sc_edge_kernel(row_hbm, col_hbm, xj_hbm, ai_hbm, aj_hbm, zeros_hbm,
                    out_hbm, agg_sh, rows, ibr, ibc, ai_l, acv, att_v,
                    sem_i, sem_l, sem_g, sem_s):
    c = lax.axis_index("c")
    s = lax.axis_index("s")
    wid = c * 16 + s
    HALF = CHUNK // 2
    rslice = pl.ds(s * ROWS_PER_SUBCORE, ROWS_PER_SUBCORE)
    pltpu.sync_copy(zeros_hbm, agg_sh.at[rslice])
    pltpu.sync_copy(ai_hbm, ai_l)

    nt = NBASE + jnp.where(wid < NREM, 1, 0)

    def chunk_off(j):
        return (wid + NTILES * j) * CHUNK

    def idx_desc(j):
        b4 = lax.rem(j, 4)
        b2 = lax.rem(j, 2)
        off = chunk_off(j)
        d0 = pltpu.make_async_copy(row_hbm.at[pl.ds(off, HALF)],
                                   ibr.at[b4, 0], sem_i.at[b4])
        d1 = pltpu.make_async_copy(row_hbm.at[pl.ds(off + HALF, HALF)],
                                   ibr.at[b4, 1], sem_i.at[b4])
        dc = pltpu.make_async_copy(col_hbm.at[pl.ds(off, CHUNK)],
                                   ibc.at[b2], sem_i.at[b4])
        return d0, d1, dc

    def logit_desc(j):
        b2 = lax.rem(j, 2)
        dc = pltpu.make_async_copy(aj_hbm.at[ibc.at[b2]], acv.at[b2],
                                   sem_l.at[b2])
        return (dc,)

    def gather_desc(j):
        b2 = lax.rem(j, 2)
        return pltpu.make_async_copy(xj_hbm.at[ibc.at[b2]], rows.at[b2],
                                     sem_g.at[b2])

    def scatter_desc(j, h):
        b2 = lax.rem(j, 2)
        b4 = lax.rem(j, 4)
        return pltpu.make_async_copy(rows.at[b2, pl.ds(h * HALF, HALF)],
                                     agg_sh.at[ibr.at[b4, h]],
                                     sem_s.at[b2])

    plsc.subcore_barrier()

    for d in idx_desc(0):
        d.start()
    for d in idx_desc(1):
        d.start()
    for d in idx_desc(0):
        d.wait()
    for d in logit_desc(0):
        d.start()
    gather_desc(0).start()

    def body(j, carry):
        b2 = lax.rem(j, 2)

        b4 = lax.rem(j, 4)
        for d in logit_desc(j):
            d.wait()
        ir_r = ibr.at[b4]
        ac_r = acv.at[b2]
        for g in range(CHUNK // 16):
            h, o = divmod(g * 16, HALF)
            r16 = ir_r[h, pl.ds(o, 16)]
            ar16 = plsc.load_gather(ai_l, [r16])
            ac16 = ac_r[pl.ds(g * 16, 16)]
            att_v[pl.ds(g * 16, 16)] = 1.0 / (1.0 + jnp.exp(-(ar16 + ac16)))

        gather_desc(j).wait()

        @pl.when(j >= 1)
        def _():
            for h in range(2):
                scatter_desc(j - 1, h).wait()

        @pl.when(j + 2 < nt)
        def _():
            for d in idx_desc(j + 2):
                d.start()

        @pl.when(j + 1 < nt)
        def _():
            for d in idx_desc(j + 1):
                d.wait()
            for d in logit_desc(j + 1):
                d.start()
            gather_desc(j + 1).start()

        rb = rows.at[b2]

        def scale_body(e, carry2):
            a16 = plsc.load_gather(att_v, [jnp.full((16,), e, jnp.int32)])
            for q in range(D // 16):
                sl = pl.ds(q * 16, 16)
                rb[e, sl] = rb[e, sl] * a16
            return carry2

        lax.fori_loop(0, HALF, scale_body, 0, unroll=8)
        scatter_desc(j, 0).start(add=True)
        lax.fori_loop(HALF, CHUNK, scale_body, 0, unroll=8)
        scatter_desc(j, 1).start(add=True)

        return carry

    lax.fori_loop(0, nt, body, 0)
    for h in range(2):
        scatter_desc(nt - 1, h).wait()

    plsc.subcore_barrier()
    pltpu.sync_copy(agg_sh.at[rslice],
                    out_hbm.at[pl.ds(c * NP + s * ROWS_PER_SUBCORE,
                                     ROWS_PER_SUBCORE)])


def _post_body(x_ref, w1t_ref, b1_ref, p_ref, wot_ref, bo_ref, g_ref, be_ref,
               y_ref):
    x = x_ref[...]
    xi = _leaky(jnp.dot(x, w1t_ref[...], preferred_element_type=jnp.float32)
                + b1_ref[...])
    agg = p_ref[0] + p_ref[1]
    agg = 1.0 / (1.0 + jnp.exp(-agg))
    out = (jnp.dot(agg, wot_ref[...], preferred_element_type=jnp.float32)
           + bo_ref[...] + xi + x)
    mean = jnp.mean(out, axis=-1, keepdims=True)
    ctr = out - mean
    var = jnp.mean(ctr * ctr, axis=-1, keepdims=True)
    y_ref[...] = ctr * lax.rsqrt(var + 1e-5) * g_ref[...] + be_ref[...]


def _run_post(x0, w1t, b1r, partials, wot, bor, g1r, be1r):
    full = lambda: pl.BlockSpec((1, D), lambda i: (0, 0))
    return pl.pallas_call(
        _post_body,
        grid=(N_BLOCKS,),
        in_specs=[
            pl.BlockSpec((ROWS_BLK, D), lambda i: (i, 0)),
            pl.BlockSpec((D, D), lambda i: (0, 0)),
            full(),
            pl.BlockSpec((2, ROWS_BLK, D), lambda i: (0, i, 0)),
            pl.BlockSpec((D, D), lambda i: (0, 0)),
            full(), full(), full(),
        ],
        out_specs=pl.BlockSpec((ROWS_BLK, D), lambda i: (i, 0)),
        out_shape=jax.ShapeDtypeStruct((N, D), jnp.float32),
    )(x0, w1t, b1r, partials, wot, bor, g1r, be1r)


def kernel(x0, x1, edge_index, W1, b1, W2, b2, a1w, a1b, a2w, a2b, Wo, bo,
           g1, be1):
    row = edge_index[0]
    col = edge_index[1]
    b1r = b1.reshape(1, D)
    b2r = b2.reshape(1, D)
    abr = jnp.concatenate([a1b, a2b, jnp.zeros((D - 2,), jnp.float32)])
    abr = abr.reshape(1, D)

    xj0, ai0, aj0 = _run_pre(x0, W1.T, b1r, W2.T, b2r, a1w, a2w, abr)

    zeros = jnp.zeros((ROWS_PER_SUBCORE, D), jnp.float32)
    partials = _sc_edge_kernel(row, col, xj0, ai0.reshape(N),
                               aj0.reshape(N), zeros)
    partials = partials.reshape(2, NP, D)[:, :N]

    return _run_post(x0, W1.T, b1r, partials, Wo.T, bo.reshape(1, D),
                     g1.reshape(1, D), be1.reshape(1, D))
